# baseline (speedup 1.0000x reference)
import functools

import jax
import jax.numpy as jnp
from jax import lax
from jax.experimental import pallas as pl
from jax.experimental.pallas import tpu as pltpu
from jax.experimental.pallas import tpu_sc as plsc

NUM_HEADS = 26
N_PER_HEAD = 100000
D = 32
BATCH = 16384
TOTAL = BATCH * NUM_HEADS
NUM_WORKERS = 32
PER_W = TOTAL // NUM_WORKERS
STEP = 128
CHUNK_STEPS = 13
CHUNK = STEP * CHUNK_STEPS
NUM_CHUNKS = PER_W // CHUNK
VECS = PER_W // 16


def _body(ids_hbm, table_hbm, out_hbm, ids_v, rows_v, sem):
    wid = lax.axis_index("s") * 2 + lax.axis_index("c")
    base = wid * PER_W
    pltpu.sync_copy(ids_hbm.at[pl.ds(base, PER_W)], ids_v)

    lane = lax.iota(jnp.int32, 16)

    def add_off(i, carry):
        pos = base + i * 16 + lane
        off = (pos % NUM_HEADS) * N_PER_HEAD
        ids_v[pl.ds(i * 16, 16)] = ids_v[pl.ds(i * 16, 16)] + off
        return carry

    lax.fori_loop(0, VECS, add_off, 0)

    def chunk_body(c, carry):
        cbase = c * CHUNK
        copies = []
        for j in range(CHUNK_STEPS):
            idx = ids_v.at[pl.ds(cbase + j * STEP, STEP)]
            dst = rows_v.at[pl.ds(j * STEP, STEP)]
            copies.append(pltpu.async_copy(table_hbm.at[idx], dst, sem))
        for cp in copies:
            cp.wait()
        pltpu.sync_copy(rows_v, out_hbm.at[pl.ds(base + cbase, CHUNK)])
        return carry

    lax.fori_loop(0, NUM_CHUNKS, chunk_body, 0)


_gather = functools.partial(
    pl.kernel,
    out_type=jax.ShapeDtypeStruct((TOTAL, D), jnp.float32),
    scratch_types=[
        pltpu.VMEM((PER_W,), jnp.int32),
        pltpu.VMEM((CHUNK, D), jnp.float32),
        pltpu.SemaphoreType.DMA,
    ],
    mesh=plsc.VectorSubcoreMesh(core_axis_name="c", subcore_axis_name="s"),
    compiler_params=pltpu.CompilerParams(use_tc_tiling_on_sc=False),
)(_body)


def kernel(input_ids, table):
    flat_ids = input_ids.reshape(-1).astype(jnp.int32)
    out = _gather(flat_ids, table)
    return out.reshape(BATCH, NUM_HEADS, D)
